# Initial kernel scaffold; baseline (speedup 1.0000x reference)
import jax
import jax.numpy as jnp
from jax.experimental import pallas as pl


def kernel(loc, vel, charges, edge_attr, y, edge_index, ptr, params):
    raise NotImplementedError("write your pallas kernel here")


# speedup vs baseline: 15.3127x; 15.3127x over previous
import functools

import jax
import jax.numpy as jnp
from jax import lax
from jax.experimental import pallas as pl
from jax.experimental.pallas import tpu as pltpu
from jax.experimental.pallas import tpu_sc as plsc

N = 50000
E = 800000
H = 64

NPAD = 50176
NHALF = 25088
GARB = 32
ACCR = NHALF + GARB
EPAD = 802816
GCH = 128
GPT = EPAD // (32 * GCH)
SPT = EPAD // (16 * GCH)
EPS = EPAD // 16
NPT = NHALF // 16
ZPT = ACCR // 16
BN = 1024
BE = 2048

_INTERPRET = False


def _silu(x):
    return x * jax.nn.sigmoid(x)


def _full(shape):
    return pl.BlockSpec(shape, lambda i: (0,) * len(shape))


def _rows(b, w):
    return pl.BlockSpec((b, w), lambda i: (i, 0))


def _embed_body(hin_ref, we1_ref, be1_ref, we2_ref, be2_ref,
                w1a_ref, w1b_ref, b1_ref,
                h0_ref, a_ref, b_ref):
    hin = hin_ref[...]
    we1 = we1_ref[...]
    hid = _silu(hin[:, 0:1] * we1[0:1, :] + hin[:, 1:2] * we1[1:2, :]
                + be1_ref[...])
    h0 = jnp.dot(hid, we2_ref[...], preferred_element_type=jnp.float32) \
        + be2_ref[...]
    h0_ref[...] = h0
    a_ref[...] = jnp.dot(h0, w1a_ref[...], preferred_element_type=jnp.float32)
    b_ref[...] = jnp.dot(h0, w1b_ref[...], preferred_element_type=jnp.float32) \
        + b1_ref[...]


def _embed_call(hin8, we1, be1, we2, be2, w1a, w1b, b1):
    grid = (NPAD // BN,)
    return pl.pallas_call(
        _embed_body,
        grid=grid,
        in_specs=[_rows(BN, 8), _full((2, H)), _full((1, H)), _full((H, H)),
                  _full((1, H)), _full((H, H)), _full((H, H)), _full((1, H))],
        out_specs=[_rows(BN, H), _rows(BN, H), _rows(BN, H)],
        out_shape=[jax.ShapeDtypeStruct((NPAD, H), jnp.float32)] * 3,
        interpret=_INTERPRET,
    )(hin8, we1, be1, we2, be2, w1a, w1b, b1)


def _edge_body(s_ref, rel_ref, ea_ref, w1c_ref, w1d_ref, w2_ref, b2_ref,
               wc1_ref, bc1_ref, wc2_ref, bc2_ref,
               m_ref, u_ref):
    s = s_ref[...]
    rel = rel_ref[...]
    ea = ea_ref[...]
    dist0 = (rel[:, 0:1] * rel[:, 0:1] + rel[:, 1:2] * rel[:, 1:2]
             + rel[:, 2:3] * rel[:, 2:3])
    dist1 = (rel[:, 3:4] * rel[:, 3:4] + rel[:, 4:5] * rel[:, 4:5]
             + rel[:, 5:6] * rel[:, 5:6])
    w1c = w1c_ref[...]
    w1d = w1d_ref[...]
    pre = (s + dist0 * w1c[0:1, :] + dist1 * w1c[1:2, :]
           + ea[:, 0:1] * w1d[0:1, :] + ea[:, 1:2] * w1d[1:2, :])
    m = jnp.dot(_silu(pre), w2_ref[...], preferred_element_type=jnp.float32) \
        + b2_ref[...]
    m_ref[...] = m
    ch = _silu(jnp.dot(m, wc1_ref[...], preferred_element_type=jnp.float32)
               + bc1_ref[...])
    w = jnp.dot(ch, wc2_ref[...], preferred_element_type=jnp.float32) \
        + bc2_ref[...]
    w0 = w[:, 0:1]
    w1 = w[:, 1:2]
    wcat = jnp.concatenate([w0, w0, w0, w1, w1, w1] + [w0] * 10, axis=1)
    u_ref[...] = rel * wcat


def _edge_call(sE, rel16, ea8, w1c, w1d, w2, b2, wc1, bc1, wc2, bc2):
    grid = (EPAD // BE,)
    return pl.pallas_call(
        _edge_body,
        grid=grid,
        in_specs=[_rows(BE, H), _rows(BE, 16), _rows(BE, 8),
                  _full((2, H)), _full((2, H)), _full((H, H)), _full((1, H)),
                  _full((H, H)), _full((1, H)), _full((H, 2)), _full((1, 2))],
        out_specs=[_rows(BE, H), _rows(BE, 16)],
        out_shape=[jax.ShapeDtypeStruct((EPAD, H), jnp.float32),
                   jax.ShapeDtypeStruct((EPAD, 16), jnp.float32)],
        interpret=_INTERPRET,
    )(sE, rel16, ea8, w1c, w1d, w2, b2, wc1, bc1, wc2, bc2)


def _node_body(h_ref, aggm_ref, cov_ref, aggu_ref, inv_ref,
               wu1a_ref, wu1b_ref, bu1_ref, wu2_ref, bu2_ref,
               w1a_ref, w1b_ref, b1_ref,
               hn_ref, covn_ref, a_ref, b_ref):
    h = h_ref[...]
    invc = inv_ref[...][:, 0:1]
    aggm = aggm_ref[...] * invc
    hid = _silu(jnp.dot(h, wu1a_ref[...], preferred_element_type=jnp.float32)
                + jnp.dot(aggm, wu1b_ref[...],
                          preferred_element_type=jnp.float32)
                + bu1_ref[...])
    hn = h + jnp.dot(hid, wu2_ref[...], preferred_element_type=jnp.float32) \
        + bu2_ref[...]
    hn_ref[...] = hn
    covn_ref[...] = cov_ref[...] + aggu_ref[...] * invc
    a_ref[...] = jnp.dot(hn, w1a_ref[...], preferred_element_type=jnp.float32)
    b_ref[...] = jnp.dot(hn, w1b_ref[...], preferred_element_type=jnp.float32) \
        + b1_ref[...]


def _node_call(h, aggM, cov16, aggU, inv16,
               wu1a, wu1b, bu1, wu2, bu2, w1a, w1b, b1):
    grid = (NPAD // BN,)
    return pl.pallas_call(
        _node_body,
        grid=grid,
        in_specs=[_rows(BN, H), _rows(BN, H), _rows(BN, 16), _rows(BN, 16),
                  _rows(BN, 16),
                  _full((H, H)), _full((H, H)), _full((1, H)), _full((H, H)),
                  _full((1, H)), _full((H, H)), _full((H, H)), _full((1, H))],
        out_specs=[_rows(BN, H), _rows(BN, 16), _rows(BN, H), _rows(BN, H)],
        out_shape=[jax.ShapeDtypeStruct((NPAD, H), jnp.float32),
                   jax.ShapeDtypeStruct((NPAD, 16), jnp.float32),
                   jax.ShapeDtypeStruct((NPAD, H), jnp.float32),
                   jax.ShapeDtypeStruct((NPAD, H), jnp.float32)],
        interpret=_INTERPRET,
    )(h, aggM, cov16, aggU, inv16, wu1a, wu1b, bu1, wu2, bu2, w1a, w1b, b1)


def _loss_body(cov_ref, loc_ref, y_ref, se_ref):
    x = cov_ref[...][:, 0:3] + loc_ref[...][:, 0:3]
    yv = y_ref[...][:, 0:3]
    mask = ~jnp.isnan(yv)
    tsafe = jnp.where(mask, yv, 0.0)
    se = jnp.where(mask, (x - tsafe) ** 2, 0.0)
    se_ref[...] = jnp.concatenate(
        [se, jnp.zeros((se.shape[0], 13), jnp.float32)], axis=1)


def _loss_call(cov16, loc16, y16):
    grid = (NPAD // BN,)
    return pl.pallas_call(
        _loss_body,
        grid=grid,
        in_specs=[_rows(BN, 16), _rows(BN, 16), _rows(BN, 16)],
        out_specs=[_rows(BN, 16)],
        out_shape=[jax.ShapeDtypeStruct((NPAD, 16), jnp.float32)],
        interpret=_INTERPRET,
    )(cov16, loc16, y16)[0]


def _sc_mesh():
    return plsc.VectorSubcoreMesh(core_axis_name="c", subcore_axis_name="s")


_SC_PARAMS = pltpu.CompilerParams(use_tc_tiling_on_sc=False)


def _sc_gather(A, B, cov16, srcP, dstP):
    @functools.partial(
        pl.kernel, mesh=_sc_mesh(), compiler_params=_SC_PARAMS,
        out_type=[jax.ShapeDtypeStruct((EPAD, H), jnp.float32),
                  jax.ShapeDtypeStruct((EPAD, 16), jnp.float32)],
        scratch_types=[
            pltpu.VMEM((GCH,), jnp.int32),
            pltpu.VMEM((GCH,), jnp.int32),
            pltpu.VMEM((GCH, H), jnp.float32),
            pltpu.VMEM((GCH, H), jnp.float32),
            pltpu.VMEM((GCH, 16), jnp.float32),
            pltpu.VMEM((GCH, 16), jnp.float32),
            pltpu.SemaphoreType.DMA,
        ],
    )
    def k(a_hbm, b_hbm, cov_hbm, src_hbm, dst_hbm, s_out, rel_out,
          idxs, idxd, bufS, bufD, bufCS, bufCD, sem):
        wid = lax.axis_index("s") * 2 + lax.axis_index("c")
        tbase = wid * (GPT * GCH)

        def chunk(i, _):
            base = tbase + i * GCH
            pltpu.sync_copy(src_hbm.at[pl.ds(base, GCH)], idxs)
            pltpu.sync_copy(dst_hbm.at[pl.ds(base, GCH)], idxd)
            ca = pltpu.async_copy(a_hbm.at[idxs], bufS, sem)
            cb = pltpu.async_copy(b_hbm.at[idxd], bufD, sem)
            cc = pltpu.async_copy(cov_hbm.at[idxs], bufCS, sem)
            cd = pltpu.async_copy(cov_hbm.at[idxd], bufCD, sem)
            ca.wait()
            cb.wait()
            cc.wait()
            cd.wait()

            def vadd(j, _):
                r = j // 4
                c = (j % 4) * 16
                bufS[r, pl.ds(c, 16)] = (bufS[r, pl.ds(c, 16)]
                                         + bufD[r, pl.ds(c, 16)])
                return 0

            lax.fori_loop(0, GCH * 4, vadd, 0, unroll=4)

            def vsub(r, _):
                bufCS[r, pl.ds(0, 16)] = (bufCS[r, pl.ds(0, 16)]
                                          - bufCD[r, pl.ds(0, 16)])
                return 0

            lax.fori_loop(0, GCH, vsub, 0, unroll=4)

            pltpu.sync_copy(bufS, s_out.at[pl.ds(base, GCH)])
            pltpu.sync_copy(bufCS, rel_out.at[pl.ds(base, GCH)])
            return 0

        lax.fori_loop(0, GPT, chunk, 0)

    return k(A, B, cov16, srcP, dstP)


def _localize_loop(idxd, cbase):
    lane = lax.iota(jnp.int32, 16)

    def body(j, _):
        v = idxd[pl.ds(j * 16, 16)]
        l = v - cbase
        ok = (l >= 0) & (l < NHALF)
        garb = NHALF + lane + (j % 2) * 16
        idxd[pl.ds(j * 16, 16)] = jnp.where(ok, l, garb)
        return 0

    lax.fori_loop(0, GCH // 16, body, 0, unroll=2)


def _zero_fill(buf, rows, width):
    z = jnp.zeros((16,), jnp.float32)

    def body(j, _):
        r = j // (width // 16)
        c = (j % (width // 16)) * 16
        buf[r, pl.ds(c, 16)] = z
        return 0

    lax.fori_loop(0, rows * (width // 16), body, 0, unroll=4)


def _zero_acc(acc, zbuf, sid, rows_per_tile):
    nfull = rows_per_tile // GCH
    tail = rows_per_tile - nfull * GCH
    base = sid * rows_per_tile

    def body(z, _):
        pltpu.sync_copy(zbuf, acc.at[pl.ds(base + z * GCH, GCH)])
        return 0

    lax.fori_loop(0, nfull, body, 0)
    if tail:
        pltpu.sync_copy(zbuf.at[pl.ds(0, tail)],
                        acc.at[pl.ds(base + nfull * GCH, tail)])


def _sc_scatter_one(vals, dstP, width):
    @functools.partial(
        pl.kernel, mesh=_sc_mesh(), compiler_params=_SC_PARAMS,
        out_type=jax.ShapeDtypeStruct((NPAD, width), jnp.float32),
        scratch_types=[
            pltpu.VMEM((GCH,), jnp.int32),
            pltpu.VMEM((GCH, width), jnp.float32),
            pltpu.VMEM((GCH, width), jnp.float32),
            pltpu.VMEM_SHARED((ACCR, width), jnp.float32),
        ],
    )
    def k(v_hbm, dst_hbm, agg_hbm, idxd, buf, zb, acc):
        c = lax.axis_index("c")
        sid = lax.axis_index("s")
        cbase = c * NHALF

        _zero_fill(zb, GCH, width)
        _zero_acc(acc, zb, sid, ZPT)
        plsc.subcore_barrier()

        def chunk(i, _):
            ebase = sid * EPS + i * GCH
            pltpu.sync_copy(dst_hbm.at[pl.ds(ebase, GCH)], idxd)
            _localize_loop(idxd, cbase)
            pltpu.sync_copy(v_hbm.at[pl.ds(ebase, GCH)], buf)
            pltpu.sync_copy(buf, acc.at[idxd], add=True)
            return 0

        lax.fori_loop(0, SPT, chunk, 0)
        plsc.subcore_barrier()

        lbase = sid * NPT
        gbase = cbase + lbase
        pltpu.sync_copy(acc.at[pl.ds(lbase, NPT)],
                        agg_hbm.at[pl.ds(gbase, NPT)])

    return k(vals, dstP)


def _sc_scatter(m, u16, dstP):
    aggM = _sc_scatter_one(m, dstP, H)
    aggU = _sc_scatter_one(u16, dstP, 16)
    return aggM, aggU


def _sc_count(dstP):
    @functools.partial(
        pl.kernel, mesh=_sc_mesh(), compiler_params=_SC_PARAMS,
        out_type=jax.ShapeDtypeStruct((NPAD, 16), jnp.float32),
        scratch_types=[
            pltpu.VMEM((GCH,), jnp.int32),
            pltpu.VMEM((GCH, 16), jnp.float32),
            pltpu.VMEM((GCH, 16), jnp.float32),
            pltpu.VMEM((NPT, 16), jnp.float32),
            pltpu.VMEM_SHARED((ACCR, 16), jnp.float32),
        ],
    )
    def k(dst_hbm, inv_hbm, idxd, ones, zbU, dbuf, accC):
        c = lax.axis_index("c")
        sid = lax.axis_index("s")
        cbase = c * NHALF

        one = jnp.ones((16,), jnp.float32)

        def fill(r, _):
            ones[r, pl.ds(0, 16)] = one
            return 0

        lax.fori_loop(0, GCH, fill, 0, unroll=4)
        _zero_fill(zbU, GCH, 16)
        _zero_acc(accC, zbU, sid, ZPT)
        plsc.subcore_barrier()

        def chunk(i, _):
            ebase = sid * EPS + i * GCH
            pltpu.sync_copy(dst_hbm.at[pl.ds(ebase, GCH)], idxd)
            _localize_loop(idxd, cbase)
            pltpu.sync_copy(ones, accC.at[idxd], add=True)
            return 0

        lax.fori_loop(0, SPT, chunk, 0)
        plsc.subcore_barrier()

        lbase = sid * NPT
        gbase = cbase + lbase
        pltpu.sync_copy(accC.at[pl.ds(lbase, NPT)], dbuf)

        def invert(r, _):
            v = dbuf[r, pl.ds(0, 16)]
            dbuf[r, pl.ds(0, 16)] = 1.0 / jnp.maximum(v, 1.0)
            return 0

        lax.fori_loop(0, NPT, invert, 0, unroll=4)
        pltpu.sync_copy(dbuf, inv_hbm.at[pl.ds(gbase, NPT)])

    return k(dstP)


def _gather_scaffold(A, B, cov16, srcP, dstP):
    sE = A[srcP] + B[dstP]
    rel16 = cov16[srcP] - cov16[dstP]
    return sE, rel16


def _scatter_scaffold(m, u16, dstP):
    aggM = jax.ops.segment_sum(m[:E], dstP[:E], num_segments=NPAD)
    aggU = jax.ops.segment_sum(u16[:E], dstP[:E], num_segments=NPAD)
    return aggM, aggU


def _count_scaffold(dstP):
    cnt = jax.ops.segment_sum(jnp.ones((E,), jnp.float32), dstP[:E],
                              num_segments=NPAD)
    inv = 1.0 / jnp.maximum(cnt, 1.0)
    return jnp.broadcast_to(inv[:, None], (NPAD, 16))


_USE_SC = True


def kernel(loc, vel, charges, edge_attr, y, edge_index, ptr, params):
    f32 = jnp.float32
    src = edge_index[0]
    dst = edge_index[1]
    srcP = jnp.concatenate([src, jnp.zeros((EPAD - E,), jnp.int32)])
    dstP = jnp.concatenate([dst, jnp.full((EPAD - E,), N, jnp.int32)])
    ea8 = jnp.zeros((EPAD, 8), f32).at[:E, 0:2].set(edge_attr)

    vnorm = jnp.sqrt(jnp.sum(vel * vel, axis=1, keepdims=True))
    hin8 = jnp.zeros((NPAD, 8), f32).at[:N, 0:1].set(vnorm)
    hin8 = hin8.at[:N, 1:2].set(charges)
    cov16 = jnp.zeros((NPAD, 16), f32).at[:N, 0:3].set(loc)
    cov16 = cov16.at[:N, 3:6].set(vel)
    loc16 = jnp.zeros((NPAD, 16), f32).at[:N, 0:3].set(loc)
    y16 = jnp.zeros((NPAD, 16), f32).at[:N, 0:3].set(y)

    emb = params['embedder']
    lps = params['layers']

    def msg_w(lp):
        W1 = lp['msg']['W1']
        return (W1[0:H], W1[H:2 * H], W1[2 * H:2 * H + 2], W1[2 * H + 2:],
                lp['msg']['b1'].reshape(1, H))

    if _USE_SC:
        inv16 = _sc_count(dstP)
    else:
        inv16 = _count_scaffold(dstP)

    w1a0, w1b0, _, _, b10 = msg_w(lps[0])
    h, A, B = _embed_call(hin8, emb['W1'], emb['b1'].reshape(1, H),
                          emb['W2'], emb['b2'].reshape(1, H),
                          w1a0, w1b0, b10)

    for li in range(4):
        lp = lps[li]
        _, _, w1c, w1d, _ = msg_w(lp)
        if _USE_SC:
            sE, rel16 = _sc_gather(A, B, cov16, srcP, dstP)
        else:
            sE, rel16 = _gather_scaffold(A, B, cov16, srcP, dstP)
        m, u16 = _edge_call(
            sE, rel16, ea8, w1c, w1d,
            lp['msg']['W2'], lp['msg']['b2'].reshape(1, H),
            lp['coord']['W1'], lp['coord']['b1'].reshape(1, H),
            lp['coord']['W2'], lp['coord']['b2'].reshape(1, 2))
        if _USE_SC:
            aggM, aggU = _sc_scatter(m, u16, dstP)
        else:
            aggM, aggU = _scatter_scaffold(m, u16, dstP)
        nw1a, nw1b, _, _, nb1 = msg_w(lps[(li + 1) % 4])
        h, cov16, A, B = _node_call(
            h, aggM, cov16, aggU, inv16,
            lp['upd']['W1'][0:H], lp['upd']['W1'][H:2 * H],
            lp['upd']['b1'].reshape(1, H),
            lp['upd']['W2'], lp['upd']['b2'].reshape(1, H),
            nw1a, nw1b, nb1)

    se16 = _loss_call(cov16, loc16, y16)
    se = se16[:N, 0:3]
    mask = ~jnp.isnan(y)
    loss = se.sum() / mask.sum()
    return loss, se


# speedup vs baseline: 16.7664x; 1.0949x over previous
import functools

import jax
import jax.numpy as jnp
from jax import lax
from jax.experimental import pallas as pl
from jax.experimental.pallas import tpu as pltpu
from jax.experimental.pallas import tpu_sc as plsc

N = 50000
E = 800000
H = 64

NPAD = 50176
NHALF = 25088
GARB = 32
ACCR = NHALF + GARB
EPAD = 802816
GCH = 128
GPT = EPAD // (32 * GCH)
SPT = EPAD // (16 * GCH)
EPS = EPAD // 16
NPT = NHALF // 16
ZPT = ACCR // 16
BN = 1024
BE = 2048

_INTERPRET = False


def _silu(x):
    return x * jax.nn.sigmoid(x)


def _full(shape):
    return pl.BlockSpec(shape, lambda i: (0,) * len(shape))


def _rows(b, w):
    return pl.BlockSpec((b, w), lambda i: (i, 0))


def _embed_body(hin_ref, we1_ref, be1_ref, we2_ref, be2_ref,
                w1a_ref, w1b_ref, b1_ref,
                h0_ref, a_ref, b_ref):
    hin = hin_ref[...]
    we1 = we1_ref[...]
    hid = _silu(hin[:, 0:1] * we1[0:1, :] + hin[:, 1:2] * we1[1:2, :]
                + be1_ref[...])
    h0 = jnp.dot(hid, we2_ref[...], preferred_element_type=jnp.float32) \
        + be2_ref[...]
    h0_ref[...] = h0
    a_ref[...] = jnp.dot(h0, w1a_ref[...], preferred_element_type=jnp.float32)
    b_ref[...] = jnp.dot(h0, w1b_ref[...], preferred_element_type=jnp.float32) \
        + b1_ref[...]


def _embed_call(hin8, we1, be1, we2, be2, w1a, w1b, b1):
    grid = (NPAD // BN,)
    return pl.pallas_call(
        _embed_body,
        grid=grid,
        in_specs=[_rows(BN, 8), _full((2, H)), _full((1, H)), _full((H, H)),
                  _full((1, H)), _full((H, H)), _full((H, H)), _full((1, H))],
        out_specs=[_rows(BN, H), _rows(BN, H), _rows(BN, H)],
        out_shape=[jax.ShapeDtypeStruct((NPAD, H), jnp.float32)] * 3,
        interpret=_INTERPRET,
    )(hin8, we1, be1, we2, be2, w1a, w1b, b1)


def _edge_body(s_ref, rel_ref, ea_ref, w1c_ref, w1d_ref, w2_ref, b2_ref,
               wc1_ref, bc1_ref, wc2_ref, bc2_ref,
               m_ref, u_ref):
    s = s_ref[...]
    rel = rel_ref[...]
    ea = ea_ref[...]
    dist0 = (rel[:, 0:1] * rel[:, 0:1] + rel[:, 1:2] * rel[:, 1:2]
             + rel[:, 2:3] * rel[:, 2:3])
    dist1 = (rel[:, 3:4] * rel[:, 3:4] + rel[:, 4:5] * rel[:, 4:5]
             + rel[:, 5:6] * rel[:, 5:6])
    w1c = w1c_ref[...]
    w1d = w1d_ref[...]
    pre = (s + dist0 * w1c[0:1, :] + dist1 * w1c[1:2, :]
           + ea[:, 0:1] * w1d[0:1, :] + ea[:, 1:2] * w1d[1:2, :])
    m = jnp.dot(_silu(pre), w2_ref[...], preferred_element_type=jnp.float32) \
        + b2_ref[...]
    m_ref[...] = m
    ch = _silu(jnp.dot(m, wc1_ref[...], preferred_element_type=jnp.float32)
               + bc1_ref[...])
    w = jnp.dot(ch, wc2_ref[...], preferred_element_type=jnp.float32) \
        + bc2_ref[...]
    w0 = w[:, 0:1]
    w1 = w[:, 1:2]
    wcat = jnp.concatenate([w0, w0, w0, w1, w1, w1] + [w0] * 10, axis=1)
    u_ref[...] = rel * wcat


def _edge_call(sE, rel16, ea8, w1c, w1d, w2, b2, wc1, bc1, wc2, bc2):
    grid = (EPAD // BE,)
    return pl.pallas_call(
        _edge_body,
        grid=grid,
        in_specs=[_rows(BE, H), _rows(BE, 16), _rows(BE, 8),
                  _full((2, H)), _full((2, H)), _full((H, H)), _full((1, H)),
                  _full((H, H)), _full((1, H)), _full((H, 2)), _full((1, 2))],
        out_specs=[_rows(BE, H), _rows(BE, 16)],
        out_shape=[jax.ShapeDtypeStruct((EPAD, H), jnp.float32),
                   jax.ShapeDtypeStruct((EPAD, 16), jnp.float32)],
        interpret=_INTERPRET,
    )(sE, rel16, ea8, w1c, w1d, w2, b2, wc1, bc1, wc2, bc2)


def _node_body(h_ref, aggm_ref, cov_ref, aggu_ref, inv_ref,
               wu1a_ref, wu1b_ref, bu1_ref, wu2_ref, bu2_ref,
               w1a_ref, w1b_ref, b1_ref,
               hn_ref, covn_ref, a_ref, b_ref):
    h = h_ref[...]
    invc = inv_ref[...][:, 0:1]
    aggm = aggm_ref[...] * invc
    hid = _silu(jnp.dot(h, wu1a_ref[...], preferred_element_type=jnp.float32)
                + jnp.dot(aggm, wu1b_ref[...],
                          preferred_element_type=jnp.float32)
                + bu1_ref[...])
    hn = h + jnp.dot(hid, wu2_ref[...], preferred_element_type=jnp.float32) \
        + bu2_ref[...]
    hn_ref[...] = hn
    covn_ref[...] = cov_ref[...] + aggu_ref[...] * invc
    a_ref[...] = jnp.dot(hn, w1a_ref[...], preferred_element_type=jnp.float32)
    b_ref[...] = jnp.dot(hn, w1b_ref[...], preferred_element_type=jnp.float32) \
        + b1_ref[...]


def _node_call(h, aggM, cov16, aggU, inv16,
               wu1a, wu1b, bu1, wu2, bu2, w1a, w1b, b1):
    grid = (NPAD // BN,)
    return pl.pallas_call(
        _node_body,
        grid=grid,
        in_specs=[_rows(BN, H), _rows(BN, H), _rows(BN, 16), _rows(BN, 16),
                  _rows(BN, 16),
                  _full((H, H)), _full((H, H)), _full((1, H)), _full((H, H)),
                  _full((1, H)), _full((H, H)), _full((H, H)), _full((1, H))],
        out_specs=[_rows(BN, H), _rows(BN, 16), _rows(BN, H), _rows(BN, H)],
        out_shape=[jax.ShapeDtypeStruct((NPAD, H), jnp.float32),
                   jax.ShapeDtypeStruct((NPAD, 16), jnp.float32),
                   jax.ShapeDtypeStruct((NPAD, H), jnp.float32),
                   jax.ShapeDtypeStruct((NPAD, H), jnp.float32)],
        interpret=_INTERPRET,
    )(h, aggM, cov16, aggU, inv16, wu1a, wu1b, bu1, wu2, bu2, w1a, w1b, b1)


def _loss_body(cov_ref, loc_ref, y_ref, se_ref):
    x = cov_ref[...][:, 0:3] + loc_ref[...][:, 0:3]
    yv = y_ref[...][:, 0:3]
    mask = ~jnp.isnan(yv)
    tsafe = jnp.where(mask, yv, 0.0)
    se = jnp.where(mask, (x - tsafe) ** 2, 0.0)
    se_ref[...] = jnp.concatenate(
        [se, jnp.zeros((se.shape[0], 13), jnp.float32)], axis=1)


def _loss_call(cov16, loc16, y16):
    grid = (NPAD // BN,)
    return pl.pallas_call(
        _loss_body,
        grid=grid,
        in_specs=[_rows(BN, 16), _rows(BN, 16), _rows(BN, 16)],
        out_specs=[_rows(BN, 16)],
        out_shape=[jax.ShapeDtypeStruct((NPAD, 16), jnp.float32)],
        interpret=_INTERPRET,
    )(cov16, loc16, y16)[0]


def _sc_mesh():
    return plsc.VectorSubcoreMesh(core_axis_name="c", subcore_axis_name="s")


_SC_PARAMS = pltpu.CompilerParams(use_tc_tiling_on_sc=False)


def _sc_gather(A, B, cov16, srcP, dstP):
    EPW = GPT * GCH

    @functools.partial(
        pl.kernel, mesh=_sc_mesh(), compiler_params=_SC_PARAMS,
        out_type=[jax.ShapeDtypeStruct((EPAD, H), jnp.float32),
                  jax.ShapeDtypeStruct((EPAD, 16), jnp.float32)],
        scratch_types=[
            pltpu.VMEM((EPW,), jnp.int32),
            pltpu.VMEM((EPW,), jnp.int32),
            pltpu.VMEM((GCH, H), jnp.float32),
            pltpu.VMEM((GCH, H), jnp.float32),
            pltpu.VMEM((GCH, 16), jnp.float32),
            pltpu.VMEM((GCH, 16), jnp.float32),
            pltpu.VMEM((GCH, H), jnp.float32),
            pltpu.VMEM((GCH, H), jnp.float32),
            pltpu.VMEM((GCH, 16), jnp.float32),
            pltpu.VMEM((GCH, 16), jnp.float32),
            pltpu.SemaphoreType.DMA,
            pltpu.SemaphoreType.DMA,
        ],
    )
    def k(a_hbm, b_hbm, cov_hbm, src_hbm, dst_hbm, s_out, rel_out,
          idxs, idxd, S0, D0, C0, F0, S1, D1, C1, F1, sem0, sem1):
        wid = lax.axis_index("s") * 2 + lax.axis_index("c")
        tbase = wid * EPW
        pltpu.sync_copy(src_hbm.at[pl.ds(tbase, EPW)], idxs)
        pltpu.sync_copy(dst_hbm.at[pl.ds(tbase, EPW)], idxd)

        def issue(j, S, D, CS, CD, sem):
            ii = idxs.at[pl.ds(j * GCH, GCH)]
            di = idxd.at[pl.ds(j * GCH, GCH)]
            pltpu.async_copy(a_hbm.at[ii], S, sem)
            pltpu.async_copy(b_hbm.at[di], D, sem)
            pltpu.async_copy(cov_hbm.at[ii], CS, sem)
            pltpu.async_copy(cov_hbm.at[di], CD, sem)

        def waitset(S, D, CS, CD, sem):
            pltpu.make_async_copy(a_hbm.at[pl.ds(0, GCH)], S, sem).wait()
            pltpu.make_async_copy(b_hbm.at[pl.ds(0, GCH)], D, sem).wait()
            pltpu.make_async_copy(cov_hbm.at[pl.ds(0, GCH)], CS, sem).wait()
            pltpu.make_async_copy(cov_hbm.at[pl.ds(0, GCH)], CD, sem).wait()

        def compute(j, S, D, CS, CD):
            base = tbase + j * GCH

            def vadd(t, _):
                r = t // 4
                c = (t % 4) * 16
                S[r, pl.ds(c, 16)] = S[r, pl.ds(c, 16)] + D[r, pl.ds(c, 16)]
                return 0

            lax.fori_loop(0, GCH * 4, vadd, 0, unroll=4)

            def vsub(r, _):
                CS[r, pl.ds(0, 16)] = (CS[r, pl.ds(0, 16)]
                                       - CD[r, pl.ds(0, 16)])
                return 0

            lax.fori_loop(0, GCH, vsub, 0, unroll=4)
            pltpu.sync_copy(S, s_out.at[pl.ds(base, GCH)])
            pltpu.sync_copy(CS, rel_out.at[pl.ds(base, GCH)])

        issue(0, S0, D0, C0, F0, sem0)

        def body(p, _):
            j0 = 2 * p
            issue(j0 + 1, S1, D1, C1, F1, sem1)
            waitset(S0, D0, C0, F0, sem0)
            compute(j0, S0, D0, C0, F0)

            @pl.when(j0 + 2 < GPT)
            def _():
                issue(j0 + 2, S0, D0, C0, F0, sem0)

            waitset(S1, D1, C1, F1, sem1)
            compute(j0 + 1, S1, D1, C1, F1)
            return 0

        lax.fori_loop(0, GPT // 2, body, 0)

    return k(A, B, cov16, srcP, dstP)


def _localize_loop(idxd, cbase):
    lane = lax.iota(jnp.int32, 16)

    def body(j, _):
        v = idxd[pl.ds(j * 16, 16)]
        l = v - cbase
        ok = (l >= 0) & (l < NHALF)
        garb = NHALF + lane + (j % 2) * 16
        idxd[pl.ds(j * 16, 16)] = jnp.where(ok, l, garb)
        return 0

    lax.fori_loop(0, GCH // 16, body, 0, unroll=2)


def _zero_fill(buf, rows, width):
    z = jnp.zeros((16,), jnp.float32)

    def body(j, _):
        r = j // (width // 16)
        c = (j % (width // 16)) * 16
        buf[r, pl.ds(c, 16)] = z
        return 0

    lax.fori_loop(0, rows * (width // 16), body, 0, unroll=4)


def _zero_acc(acc, zbuf, sid, rows_per_tile):
    nfull = rows_per_tile // GCH
    tail = rows_per_tile - nfull * GCH
    base = sid * rows_per_tile

    def body(z, _):
        pltpu.sync_copy(zbuf, acc.at[pl.ds(base + z * GCH, GCH)])
        return 0

    lax.fori_loop(0, nfull, body, 0)
    if tail:
        pltpu.sync_copy(zbuf.at[pl.ds(0, tail)],
                        acc.at[pl.ds(base + nfull * GCH, tail)])


def _sc_scatter_one(vals, dstP, width):
    @functools.partial(
        pl.kernel, mesh=_sc_mesh(), compiler_params=_SC_PARAMS,
        out_type=jax.ShapeDtypeStruct((NPAD, width), jnp.float32),
        scratch_types=[
            pltpu.VMEM((GCH,), jnp.int32),
            pltpu.VMEM((GCH, width), jnp.float32),
            pltpu.VMEM((GCH, width), jnp.float32),
            pltpu.VMEM_SHARED((ACCR, width), jnp.float32),
        ],
    )
    def k(v_hbm, dst_hbm, agg_hbm, idxd, buf, zb, acc):
        c = lax.axis_index("c")
        sid = lax.axis_index("s")
        cbase = c * NHALF

        _zero_fill(zb, GCH, width)
        _zero_acc(acc, zb, sid, ZPT)
        plsc.subcore_barrier()

        def chunk(i, _):
            ebase = sid * EPS + i * GCH
            pltpu.sync_copy(dst_hbm.at[pl.ds(ebase, GCH)], idxd)
            _localize_loop(idxd, cbase)
            pltpu.sync_copy(v_hbm.at[pl.ds(ebase, GCH)], buf)
            pltpu.sync_copy(buf, acc.at[idxd], add=True)
            return 0

        lax.fori_loop(0, SPT, chunk, 0)
        plsc.subcore_barrier()

        lbase = sid * NPT
        gbase = cbase + lbase
        pltpu.sync_copy(acc.at[pl.ds(lbase, NPT)],
                        agg_hbm.at[pl.ds(gbase, NPT)])

    return k(vals, dstP)


def _sc_scatter(m, u16, dstP):
    aggM = _sc_scatter_one(m, dstP, H)
    aggU = _sc_scatter_one(u16, dstP, 16)
    return aggM, aggU


def _sc_count(dstP):
    @functools.partial(
        pl.kernel, mesh=_sc_mesh(), compiler_params=_SC_PARAMS,
        out_type=jax.ShapeDtypeStruct((NPAD, 16), jnp.float32),
        scratch_types=[
            pltpu.VMEM((GCH,), jnp.int32),
            pltpu.VMEM((GCH, 16), jnp.float32),
            pltpu.VMEM((GCH, 16), jnp.float32),
            pltpu.VMEM((NPT, 16), jnp.float32),
            pltpu.VMEM_SHARED((ACCR, 16), jnp.float32),
        ],
    )
    def k(dst_hbm, inv_hbm, idxd, ones, zbU, dbuf, accC):
        c = lax.axis_index("c")
        sid = lax.axis_index("s")
        cbase = c * NHALF

        one = jnp.ones((16,), jnp.float32)

        def fill(r, _):
            ones[r, pl.ds(0, 16)] = one
            return 0

        lax.fori_loop(0, GCH, fill, 0, unroll=4)
        _zero_fill(zbU, GCH, 16)
        _zero_acc(accC, zbU, sid, ZPT)
        plsc.subcore_barrier()

        def chunk(i, _):
            ebase = sid * EPS + i * GCH
            pltpu.sync_copy(dst_hbm.at[pl.ds(ebase, GCH)], idxd)
            _localize_loop(idxd, cbase)
            pltpu.sync_copy(ones, accC.at[idxd], add=True)
            return 0

        lax.fori_loop(0, SPT, chunk, 0)
        plsc.subcore_barrier()

        lbase = sid * NPT
        gbase = cbase + lbase
        pltpu.sync_copy(accC.at[pl.ds(lbase, NPT)], dbuf)

        def invert(r, _):
            v = dbuf[r, pl.ds(0, 16)]
            dbuf[r, pl.ds(0, 16)] = 1.0 / jnp.maximum(v, 1.0)
            return 0

        lax.fori_loop(0, NPT, invert, 0, unroll=4)
        pltpu.sync_copy(dbuf, inv_hbm.at[pl.ds(gbase, NPT)])

    return k(dstP)


def _gather_scaffold(A, B, cov16, srcP, dstP):
    sE = A[srcP] + B[dstP]
    rel16 = cov16[srcP] - cov16[dstP]
    return sE, rel16


def _scatter_scaffold(m, u16, dstP):
    aggM = jax.ops.segment_sum(m[:E], dstP[:E], num_segments=NPAD)
    aggU = jax.ops.segment_sum(u16[:E], dstP[:E], num_segments=NPAD)
    return aggM, aggU


def _count_scaffold(dstP):
    cnt = jax.ops.segment_sum(jnp.ones((E,), jnp.float32), dstP[:E],
                              num_segments=NPAD)
    inv = 1.0 / jnp.maximum(cnt, 1.0)
    return jnp.broadcast_to(inv[:, None], (NPAD, 16))


_USE_SC = True


def kernel(loc, vel, charges, edge_attr, y, edge_index, ptr, params):
    f32 = jnp.float32
    src = edge_index[0]
    dst = edge_index[1]
    srcP = jnp.concatenate([src, jnp.zeros((EPAD - E,), jnp.int32)])
    dstP = jnp.concatenate([dst, jnp.full((EPAD - E,), N, jnp.int32)])
    ea8 = jnp.zeros((EPAD, 8), f32).at[:E, 0:2].set(edge_attr)

    vnorm = jnp.sqrt(jnp.sum(vel * vel, axis=1, keepdims=True))
    hin8 = jnp.zeros((NPAD, 8), f32).at[:N, 0:1].set(vnorm)
    hin8 = hin8.at[:N, 1:2].set(charges)
    cov16 = jnp.zeros((NPAD, 16), f32).at[:N, 0:3].set(loc)
    cov16 = cov16.at[:N, 3:6].set(vel)
    loc16 = jnp.zeros((NPAD, 16), f32).at[:N, 0:3].set(loc)
    y16 = jnp.zeros((NPAD, 16), f32).at[:N, 0:3].set(y)

    emb = params['embedder']
    lps = params['layers']

    def msg_w(lp):
        W1 = lp['msg']['W1']
        return (W1[0:H], W1[H:2 * H], W1[2 * H:2 * H + 2], W1[2 * H + 2:],
                lp['msg']['b1'].reshape(1, H))

    if _USE_SC:
        inv16 = _sc_count(dstP)
    else:
        inv16 = _count_scaffold(dstP)

    w1a0, w1b0, _, _, b10 = msg_w(lps[0])
    h, A, B = _embed_call(hin8, emb['W1'], emb['b1'].reshape(1, H),
                          emb['W2'], emb['b2'].reshape(1, H),
                          w1a0, w1b0, b10)

    for li in range(4):
        lp = lps[li]
        _, _, w1c, w1d, _ = msg_w(lp)
        if _USE_SC:
            sE, rel16 = _sc_gather(A, B, cov16, srcP, dstP)
        else:
            sE, rel16 = _gather_scaffold(A, B, cov16, srcP, dstP)
        m, u16 = _edge_call(
            sE, rel16, ea8, w1c, w1d,
            lp['msg']['W2'], lp['msg']['b2'].reshape(1, H),
            lp['coord']['W1'], lp['coord']['b1'].reshape(1, H),
            lp['coord']['W2'], lp['coord']['b2'].reshape(1, 2))
        if _USE_SC:
            aggM, aggU = _sc_scatter(m, u16, dstP)
        else:
            aggM, aggU = _scatter_scaffold(m, u16, dstP)
        nw1a, nw1b, _, _, nb1 = msg_w(lps[(li + 1) % 4])
        h, cov16, A, B = _node_call(
            h, aggM, cov16, aggU, inv16,
            lp['upd']['W1'][0:H], lp['upd']['W1'][H:2 * H],
            lp['upd']['b1'].reshape(1, H),
            lp['upd']['W2'], lp['upd']['b2'].reshape(1, H),
            nw1a, nw1b, nb1)

    se16 = _loss_call(cov16, loc16, y16)
    se = se16[:N, 0:3]
    mask = ~jnp.isnan(y)
    loss = se.sum() / mask.sum()
    return loss, se


# speedup vs baseline: 29.0355x; 1.7318x over previous
import functools

import jax
import jax.numpy as jnp
from jax import lax
from jax.experimental import pallas as pl
from jax.experimental.pallas import tpu as pltpu
from jax.experimental.pallas import tpu_sc as plsc

N = 50000
E = 800000
H = 64

NPAD = 50176
NHALF = 25088
GARB = 32
ACCR = NHALF + GARB
EPAD = 802816
GCH = 128
GPT = EPAD // (32 * GCH)
SPT = EPAD // (16 * GCH)
EPS = EPAD // 16
NPT = NHALF // 16
ZPT = ACCR // 16
BN = 1024
BE = 2048

_INTERPRET = False


def _silu(x):
    return x * jax.nn.sigmoid(x)


def _full(shape):
    return pl.BlockSpec(shape, lambda i: (0,) * len(shape))


def _rows(b, w):
    return pl.BlockSpec((b, w), lambda i: (i, 0))


def _embed_body(hin_ref, we1_ref, be1_ref, we2_ref, be2_ref,
                w1a_ref, w1b_ref, b1_ref,
                h0_ref, a_ref, b_ref):
    hin = hin_ref[...]
    we1 = we1_ref[...]
    hid = _silu(hin[:, 0:1] * we1[0:1, :] + hin[:, 1:2] * we1[1:2, :]
                + be1_ref[...])
    h0 = jnp.dot(hid, we2_ref[...], preferred_element_type=jnp.float32) \
        + be2_ref[...]
    h0_ref[...] = h0
    a_ref[...] = jnp.dot(h0, w1a_ref[...], preferred_element_type=jnp.float32)
    b_ref[...] = jnp.dot(h0, w1b_ref[...], preferred_element_type=jnp.float32) \
        + b1_ref[...]


def _embed_call(hin8, we1, be1, we2, be2, w1a, w1b, b1):
    grid = (NPAD // BN,)
    return pl.pallas_call(
        _embed_body,
        grid=grid,
        in_specs=[_rows(BN, 8), _full((2, H)), _full((1, H)), _full((H, H)),
                  _full((1, H)), _full((H, H)), _full((H, H)), _full((1, H))],
        out_specs=[_rows(BN, H), _rows(BN, H), _rows(BN, H)],
        out_shape=[jax.ShapeDtypeStruct((NPAD, H), jnp.float32)] * 3,
        interpret=_INTERPRET,
    )(hin8, we1, be1, we2, be2, w1a, w1b, b1)


def _edge_body(s_ref, rel_ref, ea_ref, k16_ref, k8_ref, w2_ref, b2_ref,
               wc1_ref, bc1_ref, wc2_ref, bc2_ref, s2_ref,
               m_ref, u_ref):
    s = s_ref[...]
    rel = rel_ref[...]
    pre = (s
           + jnp.dot(rel * rel, k16_ref[...],
                     preferred_element_type=jnp.float32)
           + jnp.dot(ea_ref[...], k8_ref[...],
                     preferred_element_type=jnp.float32))
    m = jnp.dot(_silu(pre), w2_ref[...], preferred_element_type=jnp.float32) \
        + b2_ref[...]
    m_ref[...] = m
    ch = _silu(jnp.dot(m, wc1_ref[...], preferred_element_type=jnp.float32)
               + bc1_ref[...])
    w = jnp.dot(ch, wc2_ref[...], preferred_element_type=jnp.float32) \
        + bc2_ref[...]
    u_ref[...] = rel * jnp.dot(w, s2_ref[...],
                               preferred_element_type=jnp.float32)


def _edge_call(sE, rel16, ea8, k16, k8, w2, b2, wc1, bc1, wc2, bc2, s2):
    grid = (EPAD // BE,)
    return pl.pallas_call(
        _edge_body,
        grid=grid,
        in_specs=[_rows(BE, H), _rows(BE, 16), _rows(BE, 8),
                  _full((16, H)), _full((8, H)), _full((H, H)), _full((1, H)),
                  _full((H, H)), _full((1, H)), _full((H, 2)), _full((1, 2)),
                  _full((2, 16))],
        out_specs=[_rows(BE, H), _rows(BE, 16)],
        out_shape=[jax.ShapeDtypeStruct((EPAD, H), jnp.float32),
                   jax.ShapeDtypeStruct((EPAD, 16), jnp.float32)],
        interpret=_INTERPRET,
    )(sE, rel16, ea8, k16, k8, w2, b2, wc1, bc1, wc2, bc2, s2)


def _node_body(h_ref, aggm_ref, cov_ref, aggu_ref, inv_ref,
               wu1a_ref, wu1b_ref, bu1_ref, wu2_ref, bu2_ref,
               w1a_ref, w1b_ref, b1_ref,
               hn_ref, covn_ref, a_ref, b_ref):
    h = h_ref[...]
    invc = inv_ref[...][:, 0:1]
    aggm = aggm_ref[...] * invc
    hid = _silu(jnp.dot(h, wu1a_ref[...], preferred_element_type=jnp.float32)
                + jnp.dot(aggm, wu1b_ref[...],
                          preferred_element_type=jnp.float32)
                + bu1_ref[...])
    hn = h + jnp.dot(hid, wu2_ref[...], preferred_element_type=jnp.float32) \
        + bu2_ref[...]
    hn_ref[...] = hn
    covn_ref[...] = cov_ref[...] + aggu_ref[...] * invc
    a_ref[...] = jnp.dot(hn, w1a_ref[...], preferred_element_type=jnp.float32)
    b_ref[...] = jnp.dot(hn, w1b_ref[...], preferred_element_type=jnp.float32) \
        + b1_ref[...]


def _node_call(h, aggM, cov16, aggU, inv16,
               wu1a, wu1b, bu1, wu2, bu2, w1a, w1b, b1):
    grid = (NPAD // BN,)
    return pl.pallas_call(
        _node_body,
        grid=grid,
        in_specs=[_rows(BN, H), _rows(BN, H), _rows(BN, 16), _rows(BN, 16),
                  _rows(BN, 16),
                  _full((H, H)), _full((H, H)), _full((1, H)), _full((H, H)),
                  _full((1, H)), _full((H, H)), _full((H, H)), _full((1, H))],
        out_specs=[_rows(BN, H), _rows(BN, 16), _rows(BN, H), _rows(BN, H)],
        out_shape=[jax.ShapeDtypeStruct((NPAD, H), jnp.float32),
                   jax.ShapeDtypeStruct((NPAD, 16), jnp.float32),
                   jax.ShapeDtypeStruct((NPAD, H), jnp.float32),
                   jax.ShapeDtypeStruct((NPAD, H), jnp.float32)],
        interpret=_INTERPRET,
    )(h, aggM, cov16, aggU, inv16, wu1a, wu1b, bu1, wu2, bu2, w1a, w1b, b1)


def _loss_body(cov_ref, loc_ref, y_ref, se_ref):
    x = cov_ref[...][:, 0:3] + loc_ref[...][:, 0:3]
    yv = y_ref[...][:, 0:3]
    mask = ~jnp.isnan(yv)
    tsafe = jnp.where(mask, yv, 0.0)
    se = jnp.where(mask, (x - tsafe) ** 2, 0.0)
    se_ref[...] = jnp.concatenate(
        [se, jnp.zeros((se.shape[0], 13), jnp.float32)], axis=1)


def _loss_call(cov16, loc16, y16):
    grid = (NPAD // BN,)
    return pl.pallas_call(
        _loss_body,
        grid=grid,
        in_specs=[_rows(BN, 16), _rows(BN, 16), _rows(BN, 16)],
        out_specs=[_rows(BN, 16)],
        out_shape=[jax.ShapeDtypeStruct((NPAD, 16), jnp.float32)],
        interpret=_INTERPRET,
    )(cov16, loc16, y16)[0]


def _sc_mesh():
    return plsc.VectorSubcoreMesh(core_axis_name="c", subcore_axis_name="s")


_SC_PARAMS = pltpu.CompilerParams(use_tc_tiling_on_sc=False)


def _sc_gather(A, B, cov16, srcP, dstP):
    EPW = GPT * GCH

    @functools.partial(
        pl.kernel, mesh=_sc_mesh(), compiler_params=_SC_PARAMS,
        out_type=[jax.ShapeDtypeStruct((EPAD, H), jnp.float32),
                  jax.ShapeDtypeStruct((EPAD, 16), jnp.float32)],
        scratch_types=[
            pltpu.VMEM((EPW,), jnp.int32),
            pltpu.VMEM((EPW,), jnp.int32),
            pltpu.VMEM((GCH, H), jnp.float32),
            pltpu.VMEM((GCH, H), jnp.float32),
            pltpu.VMEM((GCH, 16), jnp.float32),
            pltpu.VMEM((GCH, 16), jnp.float32),
            pltpu.VMEM((GCH, H), jnp.float32),
            pltpu.VMEM((GCH, H), jnp.float32),
            pltpu.VMEM((GCH, 16), jnp.float32),
            pltpu.VMEM((GCH, 16), jnp.float32),
            pltpu.SemaphoreType.DMA,
            pltpu.SemaphoreType.DMA,
        ],
    )
    def k(a_hbm, b_hbm, cov_hbm, src_hbm, dst_hbm, s_out, rel_out,
          idxs, idxd, S0, D0, C0, F0, S1, D1, C1, F1, sem0, sem1):
        wid = lax.axis_index("s") * 2 + lax.axis_index("c")
        tbase = wid * EPW
        pltpu.sync_copy(src_hbm.at[pl.ds(tbase, EPW)], idxs)
        pltpu.sync_copy(dst_hbm.at[pl.ds(tbase, EPW)], idxd)

        def issue(j, S, D, CS, CD, sem):
            ii = idxs.at[pl.ds(j * GCH, GCH)]
            di = idxd.at[pl.ds(j * GCH, GCH)]
            pltpu.async_copy(a_hbm.at[ii], S, sem)
            pltpu.async_copy(b_hbm.at[di], D, sem)
            pltpu.async_copy(cov_hbm.at[ii], CS, sem)
            pltpu.async_copy(cov_hbm.at[di], CD, sem)

        def waitset(S, D, CS, CD, sem):
            pltpu.make_async_copy(a_hbm.at[pl.ds(0, GCH)], S, sem).wait()
            pltpu.make_async_copy(b_hbm.at[pl.ds(0, GCH)], D, sem).wait()
            pltpu.make_async_copy(cov_hbm.at[pl.ds(0, GCH)], CS, sem).wait()
            pltpu.make_async_copy(cov_hbm.at[pl.ds(0, GCH)], CD, sem).wait()

        def compute(j, S, D, CS, CD):
            base = tbase + j * GCH

            def vadd(t, _):
                r = t // 4
                c = (t % 4) * 16
                S[r, pl.ds(c, 16)] = S[r, pl.ds(c, 16)] + D[r, pl.ds(c, 16)]
                return 0

            lax.fori_loop(0, GCH * 4, vadd, 0, unroll=4)

            def vsub(r, _):
                CS[r, pl.ds(0, 16)] = (CS[r, pl.ds(0, 16)]
                                       - CD[r, pl.ds(0, 16)])
                return 0

            lax.fori_loop(0, GCH, vsub, 0, unroll=4)
            pltpu.sync_copy(S, s_out.at[pl.ds(base, GCH)])
            pltpu.sync_copy(CS, rel_out.at[pl.ds(base, GCH)])

        issue(0, S0, D0, C0, F0, sem0)

        def body(p, _):
            j0 = 2 * p
            issue(j0 + 1, S1, D1, C1, F1, sem1)
            waitset(S0, D0, C0, F0, sem0)
            compute(j0, S0, D0, C0, F0)

            @pl.when(j0 + 2 < GPT)
            def _():
                issue(j0 + 2, S0, D0, C0, F0, sem0)

            waitset(S1, D1, C1, F1, sem1)
            compute(j0 + 1, S1, D1, C1, F1)
            return 0

        lax.fori_loop(0, GPT // 2, body, 0)

    return k(A, B, cov16, srcP, dstP)


def _localize_loop(idxd, cbase):
    lane = lax.iota(jnp.int32, 16)

    def body(j, _):
        v = idxd[pl.ds(j * 16, 16)]
        l = v - cbase
        ok = (l >= 0) & (l < NHALF)
        garb = NHALF + lane + (j % 2) * 16
        idxd[pl.ds(j * 16, 16)] = jnp.where(ok, l, garb)
        return 0

    lax.fori_loop(0, GCH // 16, body, 0, unroll=2)


def _zero_fill(buf, rows, width):
    z = jnp.zeros((16,), jnp.float32)

    def body(j, _):
        r = j // (width // 16)
        c = (j % (width // 16)) * 16
        buf[r, pl.ds(c, 16)] = z
        return 0

    lax.fori_loop(0, rows * (width // 16), body, 0, unroll=4)


def _zero_acc(acc, zbuf, sid, rows_per_tile):
    nfull = rows_per_tile // GCH
    tail = rows_per_tile - nfull * GCH
    base = sid * rows_per_tile

    def body(z, _):
        pltpu.sync_copy(zbuf, acc.at[pl.ds(base + z * GCH, GCH)])
        return 0

    lax.fori_loop(0, nfull, body, 0)
    if tail:
        pltpu.sync_copy(zbuf.at[pl.ds(0, tail)],
                        acc.at[pl.ds(base + nfull * GCH, tail)])


def _sc_scatter_one(vals, dstP, width):
    @functools.partial(
        pl.kernel, mesh=_sc_mesh(), compiler_params=_SC_PARAMS,
        out_type=jax.ShapeDtypeStruct((NPAD, width), jnp.float32),
        scratch_types=[
            pltpu.VMEM((GCH,), jnp.int32),
            pltpu.VMEM((GCH,), jnp.int32),
            pltpu.VMEM((GCH, width), jnp.float32),
            pltpu.VMEM((GCH, width), jnp.float32),
            pltpu.VMEM((GCH, width), jnp.float32),
            pltpu.VMEM_SHARED((ACCR, width), jnp.float32),
            pltpu.SemaphoreType.DMA,
            pltpu.SemaphoreType.DMA,
            pltpu.SemaphoreType.DMA,
            pltpu.SemaphoreType.DMA,
        ],
    )
    def k(v_hbm, dst_hbm, agg_hbm, i0, i1, b0, b1, zb, acc, l0, l1, s0, s1):
        c = lax.axis_index("c")
        sid = lax.axis_index("s")
        cbase = c * NHALF

        _zero_fill(zb, GCH, width)
        _zero_acc(acc, zb, sid, ZPT)
        plsc.subcore_barrier()

        def load(j, ib, vb, sem):
            eb = sid * EPS + j * GCH
            pltpu.async_copy(dst_hbm.at[pl.ds(eb, GCH)], ib, sem)
            pltpu.async_copy(v_hbm.at[pl.ds(eb, GCH)], vb, sem)

        def loadwait(ib, vb, sem):
            pltpu.make_async_copy(dst_hbm.at[pl.ds(0, GCH)], ib, sem).wait()
            pltpu.make_async_copy(v_hbm.at[pl.ds(0, GCH)], vb, sem).wait()

        load(0, i0, b0, l0)
        load(1, i1, b1, l1)

        def body(p, _):
            j0 = 2 * p
            loadwait(i0, b0, l0)
            _localize_loop(i0, cbase)
            pltpu.async_copy(b0, acc.at[i0], s0, add=True)
            loadwait(i1, b1, l1)
            _localize_loop(i1, cbase)
            pltpu.async_copy(b1, acc.at[i1], s1, add=True)
            pltpu.make_async_copy(b0, acc.at[i0], s0).wait()

            @pl.when(j0 + 2 < SPT)
            def _():
                load(j0 + 2, i0, b0, l0)

            pltpu.make_async_copy(b1, acc.at[i1], s1).wait()

            @pl.when(j0 + 3 < SPT)
            def _():
                load(j0 + 3, i1, b1, l1)

            return 0

        lax.fori_loop(0, SPT // 2, body, 0)
        plsc.subcore_barrier()

        lbase = sid * NPT
        gbase = cbase + lbase
        pltpu.sync_copy(acc.at[pl.ds(lbase, NPT)],
                        agg_hbm.at[pl.ds(gbase, NPT)])

    return k(vals, dstP)


def _sc_scatter(m, u16, dstP):
    aggM = _sc_scatter_one(m, dstP, H)
    aggU = _sc_scatter_one(u16, dstP, 16)
    return aggM, aggU


def _sc_count(dstP):
    @functools.partial(
        pl.kernel, mesh=_sc_mesh(), compiler_params=_SC_PARAMS,
        out_type=jax.ShapeDtypeStruct((NPAD, 16), jnp.float32),
        scratch_types=[
            pltpu.VMEM((GCH,), jnp.int32),
            pltpu.VMEM((GCH, 16), jnp.float32),
            pltpu.VMEM((GCH, 16), jnp.float32),
            pltpu.VMEM((NPT, 16), jnp.float32),
            pltpu.VMEM_SHARED((ACCR, 16), jnp.float32),
        ],
    )
    def k(dst_hbm, inv_hbm, idxd, ones, zbU, dbuf, accC):
        c = lax.axis_index("c")
        sid = lax.axis_index("s")
        cbase = c * NHALF

        one = jnp.ones((16,), jnp.float32)

        def fill(r, _):
            ones[r, pl.ds(0, 16)] = one
            return 0

        lax.fori_loop(0, GCH, fill, 0, unroll=4)
        _zero_fill(zbU, GCH, 16)
        _zero_acc(accC, zbU, sid, ZPT)
        plsc.subcore_barrier()

        def chunk(i, _):
            ebase = sid * EPS + i * GCH
            pltpu.sync_copy(dst_hbm.at[pl.ds(ebase, GCH)], idxd)
            _localize_loop(idxd, cbase)
            pltpu.sync_copy(ones, accC.at[idxd], add=True)
            return 0

        lax.fori_loop(0, SPT, chunk, 0)
        plsc.subcore_barrier()

        lbase = sid * NPT
        gbase = cbase + lbase
        pltpu.sync_copy(accC.at[pl.ds(lbase, NPT)], dbuf)

        def invert(r, _):
            v = dbuf[r, pl.ds(0, 16)]
            dbuf[r, pl.ds(0, 16)] = 1.0 / jnp.maximum(v, 1.0)
            return 0

        lax.fori_loop(0, NPT, invert, 0, unroll=4)
        pltpu.sync_copy(dbuf, inv_hbm.at[pl.ds(gbase, NPT)])

    return k(dstP)


def _gather_scaffold(A, B, cov16, srcP, dstP):
    sE = A[srcP] + B[dstP]
    rel16 = cov16[srcP] - cov16[dstP]
    return sE, rel16


def _scatter_scaffold(m, u16, dstP):
    aggM = jax.ops.segment_sum(m[:E], dstP[:E], num_segments=NPAD)
    aggU = jax.ops.segment_sum(u16[:E], dstP[:E], num_segments=NPAD)
    return aggM, aggU


def _count_scaffold(dstP):
    cnt = jax.ops.segment_sum(jnp.ones((E,), jnp.float32), dstP[:E],
                              num_segments=NPAD)
    inv = 1.0 / jnp.maximum(cnt, 1.0)
    return jnp.broadcast_to(inv[:, None], (NPAD, 16))


_USE_SC = True


def kernel(loc, vel, charges, edge_attr, y, edge_index, ptr, params):
    f32 = jnp.float32
    src = edge_index[0]
    dst = edge_index[1]
    srcP = jnp.concatenate([src, jnp.zeros((EPAD - E,), jnp.int32)])
    dstP = jnp.concatenate([dst, jnp.full((EPAD - E,), N, jnp.int32)])
    ea8 = jnp.zeros((EPAD, 8), f32).at[:E, 0:2].set(edge_attr)

    vnorm = jnp.sqrt(jnp.sum(vel * vel, axis=1, keepdims=True))
    hin8 = jnp.zeros((NPAD, 8), f32).at[:N, 0:1].set(vnorm)
    hin8 = hin8.at[:N, 1:2].set(charges)
    cov16 = jnp.zeros((NPAD, 16), f32).at[:N, 0:3].set(loc)
    cov16 = cov16.at[:N, 3:6].set(vel)
    loc16 = jnp.zeros((NPAD, 16), f32).at[:N, 0:3].set(loc)
    y16 = jnp.zeros((NPAD, 16), f32).at[:N, 0:3].set(y)

    emb = params['embedder']
    lps = params['layers']

    def msg_w(lp):
        W1 = lp['msg']['W1']
        return (W1[0:H], W1[H:2 * H], W1[2 * H:2 * H + 2], W1[2 * H + 2:],
                lp['msg']['b1'].reshape(1, H))

    if _USE_SC:
        inv16 = _sc_count(dstP)
    else:
        inv16 = _count_scaffold(dstP)

    w1a0, w1b0, _, _, b10 = msg_w(lps[0])
    h, A, B = _embed_call(hin8, emb['W1'], emb['b1'].reshape(1, H),
                          emb['W2'], emb['b2'].reshape(1, H),
                          w1a0, w1b0, b10)

    s2 = jnp.zeros((2, 16), f32).at[0, 0:3].set(1.0).at[1, 3:6].set(1.0)

    for li in range(4):
        lp = lps[li]
        _, _, w1c, w1d, _ = msg_w(lp)
        k16 = jnp.zeros((16, H), f32)
        k16 = k16.at[0:3, :].set(jnp.broadcast_to(w1c[0:1], (3, H)))
        k16 = k16.at[3:6, :].set(jnp.broadcast_to(w1c[1:2], (3, H)))
        k8 = jnp.zeros((8, H), f32).at[0:2, :].set(w1d)
        if _USE_SC:
            sE, rel16 = _sc_gather(A, B, cov16, srcP, dstP)
        else:
            sE, rel16 = _gather_scaffold(A, B, cov16, srcP, dstP)
        m, u16 = _edge_call(
            sE, rel16, ea8, k16, k8,
            lp['msg']['W2'], lp['msg']['b2'].reshape(1, H),
            lp['coord']['W1'], lp['coord']['b1'].reshape(1, H),
            lp['coord']['W2'], lp['coord']['b2'].reshape(1, 2), s2)
        if _USE_SC:
            aggM, aggU = _sc_scatter(m, u16, dstP)
        else:
            aggM, aggU = _scatter_scaffold(m, u16, dstP)
        nw1a, nw1b, _, _, nb1 = msg_w(lps[(li + 1) % 4])
        h, cov16, A, B = _node_call(
            h, aggM, cov16, aggU, inv16,
            lp['upd']['W1'][0:H], lp['upd']['W1'][H:2 * H],
            lp['upd']['b1'].reshape(1, H),
            lp['upd']['W2'], lp['upd']['b2'].reshape(1, H),
            nw1a, nw1b, nb1)

    se16 = _loss_call(cov16, loc16, y16)
    se = se16[:N, 0:3]
    mask = ~jnp.isnan(y)
    loss = se.sum() / mask.sum()
    return loss, se


# speedup vs baseline: 29.0531x; 1.0006x over previous
import functools

import jax
import jax.numpy as jnp
from jax import lax
from jax.experimental import pallas as pl
from jax.experimental.pallas import tpu as pltpu
from jax.experimental.pallas import tpu_sc as plsc

N = 50000
E = 800000
H = 64

NPAD = 50176
NHALF = 25088
GARB = 32
ACCR = NHALF + GARB
EPAD = 802816
GCH = 128
GPT = EPAD // (32 * GCH)
SPT = EPAD // (16 * GCH)
EPS = EPAD // 16
NPT = NHALF // 16
ZPT = ACCR // 16
BN = 1024
BE = 2048

def _silu(x):
    return x * jax.nn.sigmoid(x)


def _full(shape):
    return pl.BlockSpec(shape, lambda i: (0,) * len(shape))


def _rows(b, w):
    return pl.BlockSpec((b, w), lambda i: (i, 0))


def _embed_body(hin_ref, we1_ref, be1_ref, we2_ref, be2_ref,
                w1a_ref, w1b_ref, b1_ref,
                h0_ref, a_ref, b_ref):
    hin = hin_ref[...]
    we1 = we1_ref[...]
    hid = _silu(hin[:, 0:1] * we1[0:1, :] + hin[:, 1:2] * we1[1:2, :]
                + be1_ref[...])
    h0 = jnp.dot(hid, we2_ref[...], preferred_element_type=jnp.float32) \
        + be2_ref[...]
    h0_ref[...] = h0
    a_ref[...] = jnp.dot(h0, w1a_ref[...], preferred_element_type=jnp.float32)
    b_ref[...] = jnp.dot(h0, w1b_ref[...], preferred_element_type=jnp.float32) \
        + b1_ref[...]


def _embed_call(hin8, we1, be1, we2, be2, w1a, w1b, b1):
    grid = (NPAD // BN,)
    return pl.pallas_call(
        _embed_body,
        grid=grid,
        in_specs=[_rows(BN, 8), _full((2, H)), _full((1, H)), _full((H, H)),
                  _full((1, H)), _full((H, H)), _full((H, H)), _full((1, H))],
        out_specs=[_rows(BN, H), _rows(BN, H), _rows(BN, H)],
        out_shape=[jax.ShapeDtypeStruct((NPAD, H), jnp.float32)] * 3,
    )(hin8, we1, be1, we2, be2, w1a, w1b, b1)


def _edge_body(s_ref, rel_ref, ea_ref, k16_ref, k8_ref, w2_ref, b2_ref,
               wc1_ref, bc1_ref, wc2_ref, bc2_ref, s2_ref,
               m_ref, u_ref):
    s = s_ref[...]
    rel = rel_ref[...]
    pre = (s
           + jnp.dot(rel * rel, k16_ref[...],
                     preferred_element_type=jnp.float32)
           + jnp.dot(ea_ref[...], k8_ref[...],
                     preferred_element_type=jnp.float32))
    m = jnp.dot(_silu(pre), w2_ref[...], preferred_element_type=jnp.float32) \
        + b2_ref[...]
    m_ref[...] = m
    ch = _silu(jnp.dot(m, wc1_ref[...], preferred_element_type=jnp.float32)
               + bc1_ref[...])
    w = jnp.dot(ch, wc2_ref[...], preferred_element_type=jnp.float32) \
        + bc2_ref[...]
    u_ref[...] = rel * jnp.dot(w, s2_ref[...],
                               preferred_element_type=jnp.float32)


def _edge_call(sE, rel16, ea8, k16, k8, w2, b2, wc1, bc1, wc2, bc2, s2):
    grid = (EPAD // BE,)
    return pl.pallas_call(
        _edge_body,
        grid=grid,
        in_specs=[_rows(BE, H), _rows(BE, 16), _rows(BE, 8),
                  _full((16, H)), _full((8, H)), _full((H, H)), _full((1, H)),
                  _full((H, H)), _full((1, H)), _full((H, 2)), _full((1, 2)),
                  _full((2, 16))],
        out_specs=[_rows(BE, H), _rows(BE, 16)],
        out_shape=[jax.ShapeDtypeStruct((EPAD, H), jnp.float32),
                   jax.ShapeDtypeStruct((EPAD, 16), jnp.float32)],
    )(sE, rel16, ea8, k16, k8, w2, b2, wc1, bc1, wc2, bc2, s2)


def _node_body(h_ref, aggm_ref, cov_ref, aggu_ref, inv_ref,
               wu1a_ref, wu1b_ref, bu1_ref, wu2_ref, bu2_ref,
               w1a_ref, w1b_ref, b1_ref,
               hn_ref, covn_ref, a_ref, b_ref):
    h = h_ref[...]
    invc = inv_ref[...][:, 0:1]
    aggm = aggm_ref[...] * invc
    hid = _silu(jnp.dot(h, wu1a_ref[...], preferred_element_type=jnp.float32)
                + jnp.dot(aggm, wu1b_ref[...],
                          preferred_element_type=jnp.float32)
                + bu1_ref[...])
    hn = h + jnp.dot(hid, wu2_ref[...], preferred_element_type=jnp.float32) \
        + bu2_ref[...]
    hn_ref[...] = hn
    covn_ref[...] = cov_ref[...] + aggu_ref[...] * invc
    a_ref[...] = jnp.dot(hn, w1a_ref[...], preferred_element_type=jnp.float32)
    b_ref[...] = jnp.dot(hn, w1b_ref[...], preferred_element_type=jnp.float32) \
        + b1_ref[...]


def _node_call(h, aggM, cov16, aggU, inv16,
               wu1a, wu1b, bu1, wu2, bu2, w1a, w1b, b1):
    grid = (NPAD // BN,)
    return pl.pallas_call(
        _node_body,
        grid=grid,
        in_specs=[_rows(BN, H), _rows(BN, H), _rows(BN, 16), _rows(BN, 16),
                  _rows(BN, 16),
                  _full((H, H)), _full((H, H)), _full((1, H)), _full((H, H)),
                  _full((1, H)), _full((H, H)), _full((H, H)), _full((1, H))],
        out_specs=[_rows(BN, H), _rows(BN, 16), _rows(BN, H), _rows(BN, H)],
        out_shape=[jax.ShapeDtypeStruct((NPAD, H), jnp.float32),
                   jax.ShapeDtypeStruct((NPAD, 16), jnp.float32),
                   jax.ShapeDtypeStruct((NPAD, H), jnp.float32),
                   jax.ShapeDtypeStruct((NPAD, H), jnp.float32)],
    )(h, aggM, cov16, aggU, inv16, wu1a, wu1b, bu1, wu2, bu2, w1a, w1b, b1)


def _loss_body(cov_ref, loc_ref, y_ref, se_ref):
    x = cov_ref[...][:, 0:3] + loc_ref[...][:, 0:3]
    yv = y_ref[...][:, 0:3]
    mask = ~jnp.isnan(yv)
    tsafe = jnp.where(mask, yv, 0.0)
    se = jnp.where(mask, (x - tsafe) ** 2, 0.0)
    se_ref[...] = jnp.concatenate(
        [se, jnp.zeros((se.shape[0], 13), jnp.float32)], axis=1)


def _loss_call(cov16, loc16, y16):
    grid = (NPAD // BN,)
    return pl.pallas_call(
        _loss_body,
        grid=grid,
        in_specs=[_rows(BN, 16), _rows(BN, 16), _rows(BN, 16)],
        out_specs=[_rows(BN, 16)],
        out_shape=[jax.ShapeDtypeStruct((NPAD, 16), jnp.float32)],
    )(cov16, loc16, y16)[0]


def _sc_mesh():
    return plsc.VectorSubcoreMesh(core_axis_name="c", subcore_axis_name="s")


_SC_PARAMS = pltpu.CompilerParams(use_tc_tiling_on_sc=False)


def _sc_gather(A, B, cov16, srcP, dstP):
    EPW = GPT * GCH

    @functools.partial(
        pl.kernel, mesh=_sc_mesh(), compiler_params=_SC_PARAMS,
        out_type=[jax.ShapeDtypeStruct((EPAD, H), jnp.float32),
                  jax.ShapeDtypeStruct((EPAD, 16), jnp.float32)],
        scratch_types=[
            pltpu.VMEM((EPW,), jnp.int32),
            pltpu.VMEM((EPW,), jnp.int32),
            pltpu.VMEM((GCH, H), jnp.float32),
            pltpu.VMEM((GCH, H), jnp.float32),
            pltpu.VMEM((GCH, 16), jnp.float32),
            pltpu.VMEM((GCH, 16), jnp.float32),
            pltpu.VMEM((GCH, H), jnp.float32),
            pltpu.VMEM((GCH, H), jnp.float32),
            pltpu.VMEM((GCH, 16), jnp.float32),
            pltpu.VMEM((GCH, 16), jnp.float32),
            pltpu.SemaphoreType.DMA,
            pltpu.SemaphoreType.DMA,
        ],
    )
    def k(a_hbm, b_hbm, cov_hbm, src_hbm, dst_hbm, s_out, rel_out,
          idxs, idxd, S0, D0, C0, F0, S1, D1, C1, F1, sem0, sem1):
        wid = lax.axis_index("s") * 2 + lax.axis_index("c")
        tbase = wid * EPW
        pltpu.sync_copy(src_hbm.at[pl.ds(tbase, EPW)], idxs)
        pltpu.sync_copy(dst_hbm.at[pl.ds(tbase, EPW)], idxd)

        def issue(j, S, D, CS, CD, sem):
            ii = idxs.at[pl.ds(j * GCH, GCH)]
            di = idxd.at[pl.ds(j * GCH, GCH)]
            pltpu.async_copy(a_hbm.at[ii], S, sem)
            pltpu.async_copy(b_hbm.at[di], D, sem)
            pltpu.async_copy(cov_hbm.at[ii], CS, sem)
            pltpu.async_copy(cov_hbm.at[di], CD, sem)

        def waitset(S, D, CS, CD, sem):
            pltpu.make_async_copy(a_hbm.at[pl.ds(0, GCH)], S, sem).wait()
            pltpu.make_async_copy(b_hbm.at[pl.ds(0, GCH)], D, sem).wait()
            pltpu.make_async_copy(cov_hbm.at[pl.ds(0, GCH)], CS, sem).wait()
            pltpu.make_async_copy(cov_hbm.at[pl.ds(0, GCH)], CD, sem).wait()

        def compute(j, S, D, CS, CD):
            base = tbase + j * GCH

            def vadd(t, _):
                r = t // 4
                c = (t % 4) * 16
                S[r, pl.ds(c, 16)] = S[r, pl.ds(c, 16)] + D[r, pl.ds(c, 16)]
                return 0

            lax.fori_loop(0, GCH * 4, vadd, 0, unroll=4)

            def vsub(r, _):
                CS[r, pl.ds(0, 16)] = (CS[r, pl.ds(0, 16)]
                                       - CD[r, pl.ds(0, 16)])
                return 0

            lax.fori_loop(0, GCH, vsub, 0, unroll=4)
            pltpu.sync_copy(S, s_out.at[pl.ds(base, GCH)])
            pltpu.sync_copy(CS, rel_out.at[pl.ds(base, GCH)])

        issue(0, S0, D0, C0, F0, sem0)

        def body(p, _):
            j0 = 2 * p
            issue(j0 + 1, S1, D1, C1, F1, sem1)
            waitset(S0, D0, C0, F0, sem0)
            compute(j0, S0, D0, C0, F0)

            @pl.when(j0 + 2 < GPT)
            def _():
                issue(j0 + 2, S0, D0, C0, F0, sem0)

            waitset(S1, D1, C1, F1, sem1)
            compute(j0 + 1, S1, D1, C1, F1)
            return 0

        lax.fori_loop(0, GPT // 2, body, 0)

    return k(A, B, cov16, srcP, dstP)


def _localize_loop(idxd, cbase):
    lane = lax.iota(jnp.int32, 16)

    def body(j, _):
        v = idxd[pl.ds(j * 16, 16)]
        l = v - cbase
        ok = (l >= 0) & (l < NHALF)
        garb = NHALF + lane + (j % 2) * 16
        idxd[pl.ds(j * 16, 16)] = jnp.where(ok, l, garb)
        return 0

    lax.fori_loop(0, GCH // 16, body, 0, unroll=2)


def _zero_fill(buf, rows, width):
    z = jnp.zeros((16,), jnp.float32)

    def body(j, _):
        r = j // (width // 16)
        c = (j % (width // 16)) * 16
        buf[r, pl.ds(c, 16)] = z
        return 0

    lax.fori_loop(0, rows * (width // 16), body, 0, unroll=4)


def _zero_acc(acc, zbuf, sid, rows_per_tile):
    nfull = rows_per_tile // GCH
    tail = rows_per_tile - nfull * GCH
    base = sid * rows_per_tile

    def body(z, _):
        pltpu.sync_copy(zbuf, acc.at[pl.ds(base + z * GCH, GCH)])
        return 0

    lax.fori_loop(0, nfull, body, 0)
    if tail:
        pltpu.sync_copy(zbuf.at[pl.ds(0, tail)],
                        acc.at[pl.ds(base + nfull * GCH, tail)])


def _sc_scatter_one(vals, dstP, width):
    @functools.partial(
        pl.kernel, mesh=_sc_mesh(), compiler_params=_SC_PARAMS,
        out_type=jax.ShapeDtypeStruct((NPAD, width), jnp.float32),
        scratch_types=[
            pltpu.VMEM((GCH,), jnp.int32),
            pltpu.VMEM((GCH,), jnp.int32),
            pltpu.VMEM((GCH, width), jnp.float32),
            pltpu.VMEM((GCH, width), jnp.float32),
            pltpu.VMEM((GCH, width), jnp.float32),
            pltpu.VMEM_SHARED((ACCR, width), jnp.float32),
            pltpu.SemaphoreType.DMA,
            pltpu.SemaphoreType.DMA,
            pltpu.SemaphoreType.DMA,
            pltpu.SemaphoreType.DMA,
        ],
    )
    def k(v_hbm, dst_hbm, agg_hbm, i0, i1, b0, b1, zb, acc, l0, l1, s0, s1):
        c = lax.axis_index("c")
        sid = lax.axis_index("s")
        cbase = c * NHALF

        _zero_fill(zb, GCH, width)
        _zero_acc(acc, zb, sid, ZPT)
        plsc.subcore_barrier()

        def load(j, ib, vb, sem):
            eb = sid * EPS + j * GCH
            pltpu.async_copy(dst_hbm.at[pl.ds(eb, GCH)], ib, sem)
            pltpu.async_copy(v_hbm.at[pl.ds(eb, GCH)], vb, sem)

        def loadwait(ib, vb, sem):
            pltpu.make_async_copy(dst_hbm.at[pl.ds(0, GCH)], ib, sem).wait()
            pltpu.make_async_copy(v_hbm.at[pl.ds(0, GCH)], vb, sem).wait()

        load(0, i0, b0, l0)
        load(1, i1, b1, l1)

        def body(p, _):
            j0 = 2 * p
            loadwait(i0, b0, l0)
            _localize_loop(i0, cbase)
            pltpu.async_copy(b0, acc.at[i0], s0, add=True)
            loadwait(i1, b1, l1)
            _localize_loop(i1, cbase)
            pltpu.async_copy(b1, acc.at[i1], s1, add=True)
            pltpu.make_async_copy(b0, acc.at[i0], s0).wait()

            @pl.when(j0 + 2 < SPT)
            def _():
                load(j0 + 2, i0, b0, l0)

            pltpu.make_async_copy(b1, acc.at[i1], s1).wait()

            @pl.when(j0 + 3 < SPT)
            def _():
                load(j0 + 3, i1, b1, l1)

            return 0

        lax.fori_loop(0, SPT // 2, body, 0)
        plsc.subcore_barrier()

        lbase = sid * NPT
        gbase = cbase + lbase
        pltpu.sync_copy(acc.at[pl.ds(lbase, NPT)],
                        agg_hbm.at[pl.ds(gbase, NPT)])

    return k(vals, dstP)


def _sc_scatter(m, u16, dstP):
    aggM = _sc_scatter_one(m, dstP, H)
    aggU = _sc_scatter_one(u16, dstP, 16)
    return aggM, aggU


def _sc_count(dstP):
    @functools.partial(
        pl.kernel, mesh=_sc_mesh(), compiler_params=_SC_PARAMS,
        out_type=jax.ShapeDtypeStruct((NPAD, 16), jnp.float32),
        scratch_types=[
            pltpu.VMEM((GCH,), jnp.int32),
            pltpu.VMEM((GCH, 16), jnp.float32),
            pltpu.VMEM((GCH, 16), jnp.float32),
            pltpu.VMEM((NPT, 16), jnp.float32),
            pltpu.VMEM_SHARED((ACCR, 16), jnp.float32),
        ],
    )
    def k(dst_hbm, inv_hbm, idxd, ones, zbU, dbuf, accC):
        c = lax.axis_index("c")
        sid = lax.axis_index("s")
        cbase = c * NHALF

        one = jnp.ones((16,), jnp.float32)

        def fill(r, _):
            ones[r, pl.ds(0, 16)] = one
            return 0

        lax.fori_loop(0, GCH, fill, 0, unroll=4)
        _zero_fill(zbU, GCH, 16)
        _zero_acc(accC, zbU, sid, ZPT)
        plsc.subcore_barrier()

        def chunk(i, _):
            ebase = sid * EPS + i * GCH
            pltpu.sync_copy(dst_hbm.at[pl.ds(ebase, GCH)], idxd)
            _localize_loop(idxd, cbase)
            pltpu.sync_copy(ones, accC.at[idxd], add=True)
            return 0

        lax.fori_loop(0, SPT, chunk, 0)
        plsc.subcore_barrier()

        lbase = sid * NPT
        gbase = cbase + lbase
        pltpu.sync_copy(accC.at[pl.ds(lbase, NPT)], dbuf)

        def invert(r, _):
            v = dbuf[r, pl.ds(0, 16)]
            dbuf[r, pl.ds(0, 16)] = 1.0 / jnp.maximum(v, 1.0)
            return 0

        lax.fori_loop(0, NPT, invert, 0, unroll=4)
        pltpu.sync_copy(dbuf, inv_hbm.at[pl.ds(gbase, NPT)])

    return k(dstP)


def kernel(loc, vel, charges, edge_attr, y, edge_index, ptr, params):
    f32 = jnp.float32
    src = edge_index[0]
    dst = edge_index[1]
    srcP = jnp.concatenate([src, jnp.zeros((EPAD - E,), jnp.int32)])
    dstP = jnp.concatenate([dst, jnp.full((EPAD - E,), N, jnp.int32)])
    ea8 = jnp.zeros((EPAD, 8), f32).at[:E, 0:2].set(edge_attr)

    vnorm = jnp.sqrt(jnp.sum(vel * vel, axis=1, keepdims=True))
    hin8 = jnp.zeros((NPAD, 8), f32).at[:N, 0:1].set(vnorm)
    hin8 = hin8.at[:N, 1:2].set(charges)
    cov16 = jnp.zeros((NPAD, 16), f32).at[:N, 0:3].set(loc)
    cov16 = cov16.at[:N, 3:6].set(vel)
    loc16 = jnp.zeros((NPAD, 16), f32).at[:N, 0:3].set(loc)
    y16 = jnp.zeros((NPAD, 16), f32).at[:N, 0:3].set(y)

    emb = params['embedder']
    lps = params['layers']

    def msg_w(lp):
        W1 = lp['msg']['W1']
        return (W1[0:H], W1[H:2 * H], W1[2 * H:2 * H + 2], W1[2 * H + 2:],
                lp['msg']['b1'].reshape(1, H))

    inv16 = _sc_count(dstP)

    w1a0, w1b0, _, _, b10 = msg_w(lps[0])
    h, A, B = _embed_call(hin8, emb['W1'], emb['b1'].reshape(1, H),
                          emb['W2'], emb['b2'].reshape(1, H),
                          w1a0, w1b0, b10)

    s2 = jnp.zeros((2, 16), f32).at[0, 0:3].set(1.0).at[1, 3:6].set(1.0)

    for li in range(4):
        lp = lps[li]
        _, _, w1c, w1d, _ = msg_w(lp)
        k16 = jnp.zeros((16, H), f32)
        k16 = k16.at[0:3, :].set(jnp.broadcast_to(w1c[0:1], (3, H)))
        k16 = k16.at[3:6, :].set(jnp.broadcast_to(w1c[1:2], (3, H)))
        k8 = jnp.zeros((8, H), f32).at[0:2, :].set(w1d)
        sE, rel16 = _sc_gather(A, B, cov16, srcP, dstP)
        m, u16 = _edge_call(
            sE, rel16, ea8, k16, k8,
            lp['msg']['W2'], lp['msg']['b2'].reshape(1, H),
            lp['coord']['W1'], lp['coord']['b1'].reshape(1, H),
            lp['coord']['W2'], lp['coord']['b2'].reshape(1, 2), s2)
        aggM, aggU = _sc_scatter(m, u16, dstP)
        nw1a, nw1b, _, _, nb1 = msg_w(lps[(li + 1) % 4])
        h, cov16, A, B = _node_call(
            h, aggM, cov16, aggU, inv16,
            lp['upd']['W1'][0:H], lp['upd']['W1'][H:2 * H],
            lp['upd']['b1'].reshape(1, H),
            lp['upd']['W2'], lp['upd']['b2'].reshape(1, H),
            nw1a, nw1b, nb1)

    se16 = _loss_call(cov16, loc16, y16)
    se = se16[:N, 0:3]
    mask = ~jnp.isnan(y)
    loss = se.sum() / mask.sum()
    return loss, se
